# baseline (speedup 1.0000x reference)
import functools

import jax
import jax.numpy as jnp
import numpy as np
from jax.experimental import pallas as pl

N = 20000
NUM_CLASSES = 81
SCORE_THRESH = 0.05
NMS_THRESH = 0.5
DETS_PER_IMG = 100
PRE_NMS_TOPK = 200
IMG_H, IMG_W = 800, 1333
BBOX_XFORM_CLIP = float(np.log(1000.0 / 16.0))
WX, WY, WW, WH = 10.0, 10.0, 5.0, 5.0
NEG = jnp.float32(-1.0)


def _softmax_body(logits_ref, out_ref):
    x = logits_ref[...]
    m = jnp.max(x, axis=-1, keepdims=True)
    e = jnp.exp(x - m)
    out_ref[...] = e / jnp.sum(e, axis=-1, keepdims=True)


def _softmax_pallas(x):
    return pl.pallas_call(
        _softmax_body,
        out_shape=jax.ShapeDtypeStruct(x.shape, x.dtype),
    )(x)


def _decode(deltas, anchors):
    widths = anchors[:, 2] - anchors[:, 0] + 1.0
    heights = anchors[:, 3] - anchors[:, 1] + 1.0
    ctr_x = anchors[:, 0] + 0.5 * widths
    ctr_y = anchors[:, 1] + 0.5 * heights
    dx = deltas[:, 0::4] / WX
    dy = deltas[:, 1::4] / WY
    dw = jnp.minimum(deltas[:, 2::4] / WW, BBOX_XFORM_CLIP)
    dh = jnp.minimum(deltas[:, 3::4] / WH, BBOX_XFORM_CLIP)
    pcx = dx * widths[:, None] + ctr_x[:, None]
    pcy = dy * heights[:, None] + ctr_y[:, None]
    pw = jnp.exp(dw) * widths[:, None]
    ph = jnp.exp(dh) * heights[:, None]
    x1 = pcx - 0.5 * pw
    y1 = pcy - 0.5 * ph
    x2 = pcx + 0.5 * pw - 1.0
    y2 = pcy + 0.5 * ph - 1.0
    return jnp.stack([x1, y1, x2, y2], axis=2)


def _pairwise_iou(b):
    area = (b[:, 2] - b[:, 0] + 1.0) * (b[:, 3] - b[:, 1] + 1.0)
    lt = jnp.maximum(b[:, None, :2], b[None, :, :2])
    rb = jnp.minimum(b[:, None, 2:], b[None, :, 2:])
    wh = jnp.maximum(rb - lt + 1.0, 0.0)
    inter = wh[..., 0] * wh[..., 1]
    return inter / (area[:, None] + area[None, :] - inter)


def _nms_one_class(boxes_c, scores_c):
    valid = scores_c > SCORE_THRESH
    masked = jnp.where(valid, scores_c, NEG)
    top_scores, top_idx = jax.lax.top_k(masked, PRE_NMS_TOPK)
    top_boxes = boxes_c[top_idx]
    top_valid = top_scores > SCORE_THRESH
    iou = _pairwise_iou(jax.lax.stop_gradient(top_boxes))
    idx = jnp.arange(PRE_NMS_TOPK)

    def step(keep, i):
        suppress = (iou[i] > NMS_THRESH) & (idx > i) & keep[i]
        return keep & (~suppress), None

    keep, _ = jax.lax.scan(step, top_valid, jnp.arange(PRE_NMS_TOPK))
    out_scores = jnp.where(keep, top_scores, NEG)
    return top_boxes, out_scores


def kernel(cls_logits, bbox_reg, boxes):
    probs = _softmax_pallas(cls_logits)
    proposals = _decode(bbox_reg, boxes)
    x1 = jnp.clip(proposals[..., 0], 0.0, IMG_W - 1.0)
    y1 = jnp.clip(proposals[..., 1], 0.0, IMG_H - 1.0)
    x2 = jnp.clip(proposals[..., 2], 0.0, IMG_W - 1.0)
    y2 = jnp.clip(proposals[..., 3], 0.0, IMG_H - 1.0)
    proposals = jnp.stack([x1, y1, x2, y2], axis=-1)
    boxes_cls = jnp.transpose(proposals[:, 1:, :], (1, 0, 2))
    scores_cls = jnp.transpose(probs[:, 1:], (1, 0))
    nms_boxes, nms_scores = jax.vmap(_nms_one_class)(boxes_cls, scores_cls)
    labels = jnp.broadcast_to(jnp.arange(1, NUM_CLASSES)[:, None], nms_scores.shape)
    flat_scores = nms_scores.reshape(-1)
    flat_boxes = nms_boxes.reshape(-1, 4)
    flat_labels = labels.reshape(-1)
    final_scores, sel = jax.lax.top_k(flat_scores, DETS_PER_IMG)
    final_boxes = flat_boxes[sel]
    final_labels = flat_labels[sel]
    dets = jnp.concatenate([final_boxes, final_scores[:, None]], axis=1)
    return dets, final_labels
